# Initial kernel scaffold; baseline (speedup 1.0000x reference)
import jax
import jax.numpy as jnp
from jax.experimental import pallas as pl


def kernel(x, geom, depth_kept):
    raise NotImplementedError("write your pallas kernel here")


# speedup vs baseline: 1.8666x; 1.8666x over previous
import functools

import jax
import jax.numpy as jnp
import numpy as np
from jax import lax
from jax.experimental import pallas as pl
from jax.experimental.pallas import tpu as pltpu
from jax.experimental.pallas import tpu_sc as plsc

B, N, D, H, W, C = 1, 6, 118, 16, 44, 80
XB = (-51.2, 51.2, 0.8)
YB = (-51.2, 51.2, 0.8)
ZB = (-10.0, 10.0, 20.0)
NX, NY, NZ = 128, 128, 1
NPRIME = B * N * D * H * W

CP = 128
P = 128
NCHUNKS = NPRIME // P
NSUB = 16
ROUNDS = -(-NCHUNKS // NSUB)

HALF = (NX * NY) // 2
GRID_ROWS = HALF + 256
ZCHUNKS = GRID_ROWS // P
VROWS_PER_TILE = HALF // NSUB
DCHUNKS = VROWS_PER_TILE // P

LANES = 16
GROUPS = P // LANES


def _f32(x):
    return np.float32(x)


def _axis_consts(bounds):
    lo, hi, dx = _f32(bounds[0]), _f32(bounds[1]), _f32(bounds[2])
    span = _f32(hi - lo)
    off = _f32(lo - _f32(_f32(0.05) * span))
    bx = _f32(lo + _f32(dx / _f32(2.0)))
    sub = _f32(bx - _f32(dx / _f32(2.0)))
    return span, off, sub, dx


def _sc_kernel_body(x_hbm, meta_hbm, zeros_hbm, out_hbm,
                    meta_v, idx_v, ramp_v, xbuf_v, sem, grid_sh):
    cid_core = lax.axis_index("c")
    sid = lax.axis_index("s")

    sxc, oxc, bxc, dxc = _axis_consts(XB)
    syc, oyc, byc, dyc = _axis_consts(YB)
    szc, ozc, bzc, dzc = _axis_consts(ZB)

    iota16 = lax.iota(jnp.int32, LANES)
    row_lo = cid_core * HALF

    def build_ramp(base):
        for j in range(GROUPS):
            ramp_v[pl.ds(j * LANES, LANES)] = base + j * LANES + iota16

    pltpu.sync_copy(zeros_hbm, xbuf_v)
    for i in range(-(-ZCHUNKS // NSUB)):
        zc = i * NSUB + sid

        @pl.when(zc < ZCHUNKS)
        def _():
            build_ramp(zc * P)
            pltpu.sync_copy(xbuf_v, grid_sh.at[ramp_v])
    plsc.subcore_barrier()

    def chunk_body(k, carry):
        cid = k * NSUB + sid

        @pl.when(cid < NCHUNKS)
        def _():
            base = cid * P
            cp = pltpu.async_copy(x_hbm.at[pl.ds(base, P)], xbuf_v, sem)
            pltpu.sync_copy(meta_hbm.at[cid], meta_v)
            for j in range(GROUPS):
                sl = pl.ds(j * LANES, LANES)
                gx = meta_v[0, sl]
                gy = meta_v[1, sl]
                gz = meta_v[2, sl]
                dk = meta_v[3, sl]
                qx = ((gx * sxc) * np.float32(1.1) + oxc - bxc) / dxc
                qy = ((gy * syc) * np.float32(1.1) + oyc - byc) / dyc
                qz = ((gz * szc) * np.float32(1.1) + ozc - bzc) / dzc
                kept = ((qx >= 0.0) & (qx < np.float32(NX))
                        & (qy >= 0.0) & (qy < np.float32(NY))
                        & (qz >= 0.0) & (qz < np.float32(NZ))
                        & (dk > 0.5))
                vx = qx.astype(jnp.int32)
                vy = qy.astype(jnp.int32)
                dst = vx * NY + vy
                ldst = dst - row_lo
                mine = kept & (ldst >= 0) & (ldst < HALF)
                idx_v[sl] = jnp.where(mine, ldst, HALF + sid)
            cp.wait()
            pltpu.sync_copy(xbuf_v, grid_sh.at[idx_v], add=True)
        return carry

    lax.fori_loop(0, ROUNDS, chunk_body, 0)
    plsc.subcore_barrier()
    for t in range(DCHUNKS):
        rlo = sid * VROWS_PER_TILE + t * P
        build_ramp(rlo)
        pltpu.sync_copy(grid_sh.at[ramp_v], xbuf_v)
        pltpu.sync_copy(xbuf_v, out_hbm.at[cid_core, pl.ds(rlo, P)])


@functools.partial(jax.jit, static_argnames=())
def _sc_scatter(xp, meta, zeros):
    mesh = plsc.VectorSubcoreMesh(core_axis_name="c", subcore_axis_name="s")
    fn = pl.kernel(
        _sc_kernel_body,
        out_type=jax.ShapeDtypeStruct((2, HALF, CP), jnp.float32),
        mesh=mesh,
        scratch_types=[
            pltpu.VMEM((4, P), jnp.float32),
            pltpu.VMEM((P,), jnp.int32),
            pltpu.VMEM((P,), jnp.int32),
            pltpu.VMEM((P, CP), jnp.float32),
            pltpu.SemaphoreType.DMA,
            pltpu.VMEM_SHARED((GRID_ROWS, CP), jnp.float32),
        ],
    )
    return fn(xp, meta, zeros)


def _tc_merge_body(p_ref, o_ref):
    t = p_ref[0]
    o_ref[...] = t[:, :C].T


def _tc_merge(partials):
    BLK = 128
    nblk = (NX * NY) // BLK
    hblk = HALF // BLK
    return pl.pallas_call(
        _tc_merge_body,
        grid=(nblk,),
        in_specs=[pl.BlockSpec((1, BLK, CP),
                               lambda i: (i // hblk, i % hblk, 0))],
        out_specs=pl.BlockSpec((C, BLK), lambda i: (0, i)),
        out_shape=jax.ShapeDtypeStruct((C, NX * NY), jnp.float32),
    )(partials)


def kernel(x, geom, depth_kept):
    xp = jnp.pad(x.reshape(NPRIME, C), ((0, 0), (0, CP - C)))
    geom_t = geom.reshape(NPRIME, 3).T
    dk = depth_kept.reshape(1, NPRIME).astype(jnp.float32)
    meta = jnp.concatenate([geom_t, dk], axis=0)
    meta = meta.reshape(4, NCHUNKS, P).transpose(1, 0, 2)
    zeros = jnp.zeros((P, CP), jnp.float32)
    partials = _sc_scatter(xp, meta, zeros)
    out = _tc_merge(partials)
    return out.reshape(B, NZ * C, NX, NY)


# speedup vs baseline: 1.8675x; 1.0005x over previous
import functools

import jax
import jax.numpy as jnp
import numpy as np
from jax import lax
from jax.experimental import pallas as pl
from jax.experimental.pallas import tpu as pltpu
from jax.experimental.pallas import tpu_sc as plsc

B, N, D, H, W, C = 1, 6, 118, 16, 44, 80
XB = (-51.2, 51.2, 0.8)
YB = (-51.2, 51.2, 0.8)
ZB = (-10.0, 10.0, 20.0)
NX, NY, NZ = 128, 128, 1
NPRIME = B * N * D * H * W

CP = 128
P = 128
NCHUNKS = NPRIME // P
NSUB = 16
ROUNDS = -(-NCHUNKS // NSUB)

HALF = (NX * NY) // 2
GRID_ROWS = HALF + 256
ZCHUNKS = GRID_ROWS // P
VROWS_PER_TILE = HALF // NSUB
DCHUNKS = VROWS_PER_TILE // P

LANES = 16
GROUPS = P // LANES


def _f32(x):
    return np.float32(x)


def _axis_consts(bounds):
    lo, hi, dx = _f32(bounds[0]), _f32(bounds[1]), _f32(bounds[2])
    span = _f32(hi - lo)
    off = _f32(lo - _f32(_f32(0.05) * span))
    bx = _f32(lo + _f32(dx / _f32(2.0)))
    sub = _f32(bx - _f32(dx / _f32(2.0)))
    return span, off, sub, dx


def _sc_kernel_body(x_hbm, meta_hbm, zeros_hbm, out_hbm,
                    meta_v, idx_v, ramp_v, xbuf_v, sem, grid_sh):
    cid_core = lax.axis_index("c")
    sid = lax.axis_index("s")

    sxc, oxc, bxc, dxc = _axis_consts(XB)
    syc, oyc, byc, dyc = _axis_consts(YB)
    szc, ozc, bzc, dzc = _axis_consts(ZB)

    iota16 = lax.iota(jnp.int32, LANES)
    row_lo = cid_core * HALF

    def build_ramp(base):
        for j in range(GROUPS):
            ramp_v[pl.ds(j * LANES, LANES)] = base + j * LANES + iota16

    pltpu.sync_copy(zeros_hbm, xbuf_v)
    for i in range(-(-ZCHUNKS // NSUB)):
        zc = i * NSUB + sid

        @pl.when(zc < ZCHUNKS)
        def _():
            build_ramp(zc * P)
            pltpu.sync_copy(xbuf_v, grid_sh.at[ramp_v])
    plsc.subcore_barrier()

    def chunk_body(k, carry):
        cid = k * NSUB + sid

        @pl.when(cid < NCHUNKS)
        def _():
            cp = pltpu.async_copy(x_hbm.at[cid], xbuf_v, sem)
            pltpu.sync_copy(meta_hbm.at[cid], meta_v)
            for j in range(GROUPS):
                sl = pl.ds(j * LANES, LANES)
                gx = meta_v[0, sl]
                gy = meta_v[1, sl]
                gz = meta_v[2, sl]
                dk = meta_v[3, sl]
                qx = ((gx * sxc) * np.float32(1.1) + oxc - bxc) / dxc
                qy = ((gy * syc) * np.float32(1.1) + oyc - byc) / dyc
                qz = ((gz * szc) * np.float32(1.1) + ozc - bzc) / dzc
                kept = ((qx >= 0.0) & (qx < np.float32(NX))
                        & (qy >= 0.0) & (qy < np.float32(NY))
                        & (qz >= 0.0) & (qz < np.float32(NZ))
                        & (dk > 0.5))
                vx = qx.astype(jnp.int32)
                vy = qy.astype(jnp.int32)
                dst = vx * NY + vy
                ldst = dst - row_lo
                mine = kept & (ldst >= 0) & (ldst < HALF)
                idx_v[sl] = jnp.where(mine, ldst, HALF + sid)
            cp.wait()
            pltpu.sync_copy(xbuf_v, grid_sh.at[idx_v], add=True)
        return carry

    lax.fori_loop(0, ROUNDS, chunk_body, 0)
    plsc.subcore_barrier()
    for t in range(DCHUNKS):
        rlo = sid * VROWS_PER_TILE + t * P
        build_ramp(rlo)
        pltpu.sync_copy(grid_sh.at[ramp_v], xbuf_v)
        pltpu.sync_copy(xbuf_v, out_hbm.at[cid_core, pl.ds(rlo, P)])


@functools.partial(jax.jit, static_argnames=())
def _sc_scatter(xp, meta, zeros):
    mesh = plsc.VectorSubcoreMesh(core_axis_name="c", subcore_axis_name="s")
    fn = pl.kernel(
        _sc_kernel_body,
            out_type=jax.ShapeDtypeStruct((2, HALF, CP), jnp.float32),
        mesh=mesh,
        scratch_types=[
            pltpu.VMEM((4, P), jnp.float32),
            pltpu.VMEM((P,), jnp.int32),
            pltpu.VMEM((P,), jnp.int32),
            pltpu.VMEM((P, CP), jnp.float32),
            pltpu.SemaphoreType.DMA,
            pltpu.VMEM_SHARED((GRID_ROWS, CP), jnp.float32),
        ],
    )
    return fn(xp, meta, zeros)


def _tc_merge_body(p_ref, o_ref):
    t = p_ref[0]
    o_ref[...] = t[:, :C].T


def _tc_merge(partials):
    BLK = 128
    nblk = (NX * NY) // BLK
    hblk = HALF // BLK
    return pl.pallas_call(
        _tc_merge_body,
        grid=(nblk,),
        in_specs=[pl.BlockSpec((1, BLK, CP),
                               lambda i: (i // hblk, i % hblk, 0))],
        out_specs=pl.BlockSpec((C, BLK), lambda i: (0, i)),
        out_shape=jax.ShapeDtypeStruct((C, NX * NY), jnp.float32),
    )(partials)


def kernel(x, geom, depth_kept):
    xp = jnp.pad(x.reshape(NCHUNKS, P, C), ((0, 0), (0, 0), (0, CP - C)))
    geom_t = geom.reshape(NPRIME, 3).T
    dk = depth_kept.reshape(1, NPRIME).astype(jnp.float32)
    meta = jnp.concatenate([geom_t, dk], axis=0)
    meta = meta.reshape(4, NCHUNKS, P).transpose(1, 0, 2)
    zeros = jnp.zeros((P, CP), jnp.float32)
    partials = _sc_scatter(xp, meta, zeros)
    out = _tc_merge(partials)
    return out.reshape(B, NZ * C, NX, NY)


# speedup vs baseline: 2.1217x; 1.1361x over previous
import functools

import jax
import jax.numpy as jnp
import numpy as np
from jax import lax
from jax.experimental import pallas as pl
from jax.experimental.pallas import tpu as pltpu
from jax.experimental.pallas import tpu_sc as plsc

B, N, D, H, W, C = 1, 6, 118, 16, 44, 80
XB = (-51.2, 51.2, 0.8)
YB = (-51.2, 51.2, 0.8)
ZB = (-10.0, 10.0, 20.0)
NX, NY, NZ = 128, 128, 1
NPRIME = B * N * D * H * W

CP = 128
P = 128
NCHUNKS = NPRIME // P
NSUB = 16
ROUNDS = -(-NCHUNKS // NSUB)

HALF = (NX * NY) // 2
GRID_ROWS = HALF + 256
ZCHUNKS = GRID_ROWS // P
VROWS_PER_TILE = HALF // NSUB
DCHUNKS = VROWS_PER_TILE // P

LANES = 16
GROUPS = P // LANES


def _f32(x):
    return np.float32(x)


def _axis_consts(bounds):
    lo, hi, dx = _f32(bounds[0]), _f32(bounds[1]), _f32(bounds[2])
    span = _f32(hi - lo)
    off = _f32(lo - _f32(_f32(0.05) * span))
    bx = _f32(lo + _f32(dx / _f32(2.0)))
    sub = _f32(bx - _f32(dx / _f32(2.0)))
    return span, off, sub, dx


def _sc_kernel_body(x_hbm, meta_hbm, zeros_hbm, out_hbm,
                    meta_v, idx_v, ramp_v, xbuf_v, xbuf2_v, sem, sem2,
                    grid_sh):
    cid_core = lax.axis_index("c")
    sid = lax.axis_index("s")

    sxc, oxc, bxc, dxc = _axis_consts(XB)
    syc, oyc, byc, dyc = _axis_consts(YB)
    szc, ozc, bzc, dzc = _axis_consts(ZB)

    iota16 = lax.iota(jnp.int32, LANES)
    row_lo = cid_core * HALF

    def build_ramp(base):
        for j in range(GROUPS):
            ramp_v[pl.ds(j * LANES, LANES)] = base + j * LANES + iota16

    pltpu.sync_copy(zeros_hbm, xbuf_v)
    for i in range(-(-ZCHUNKS // NSUB)):
        zc = i * NSUB + sid

        @pl.when(zc < ZCHUNKS)
        def _():
            build_ramp(zc * P)
            pltpu.sync_copy(xbuf_v, grid_sh.at[ramp_v])
    plsc.subcore_barrier()

    def compute_idx(cid):
        pltpu.sync_copy(meta_hbm.at[cid], meta_v)
        for j in range(GROUPS):
            sl = pl.ds(j * LANES, LANES)
            gx = meta_v[0, sl]
            gy = meta_v[1, sl]
            gz = meta_v[2, sl]
            dk = meta_v[3, sl]
            qx = ((gx * sxc) * np.float32(1.1) + oxc - bxc) / dxc
            qy = ((gy * syc) * np.float32(1.1) + oyc - byc) / dyc
            qz = ((gz * szc) * np.float32(1.1) + ozc - bzc) / dzc
            kept = ((qx >= 0.0) & (qx < np.float32(NX))
                    & (qy >= 0.0) & (qy < np.float32(NY))
                    & (qz >= 0.0) & (qz < np.float32(NZ))
                    & (dk > 0.5))
            vx = qx.astype(jnp.int32)
            vy = qy.astype(jnp.int32)
            dst = vx * NY + vy
            ldst = dst - row_lo
            mine = kept & (ldst >= 0) & (ldst < HALF)
            idx_v[sl] = jnp.where(mine, ldst, HALF + sid)

    bufs = (xbuf_v, xbuf2_v)
    sems = (sem, sem2)

    def wait_gather(par, cid):
        pltpu.make_async_copy(x_hbm.at[cid], bufs[par], sems[par]).wait()

    pltpu.async_copy(x_hbm.at[sid], bufs[0], sems[0])

    def chunk_body(k2, carry):
        for par in range(2):
            k = k2 * 2 + par
            cid = k * NSUB + sid
            nxt = (k + 1) * NSUB + sid

            @pl.when(nxt < NCHUNKS)
            def _():
                pltpu.async_copy(x_hbm.at[nxt], bufs[1 - par], sems[1 - par])

            @pl.when(cid < NCHUNKS)
            def _():
                compute_idx(cid)
                wait_gather(par, cid)
                pltpu.sync_copy(bufs[par], grid_sh.at[idx_v], add=True)
        return carry

    lax.fori_loop(0, ROUNDS // 2, chunk_body, 0)
    plsc.subcore_barrier()
    for t in range(DCHUNKS):
        rlo = sid * VROWS_PER_TILE + t * P
        build_ramp(rlo)
        pltpu.sync_copy(grid_sh.at[ramp_v], xbuf_v)
        pltpu.sync_copy(xbuf_v, out_hbm.at[cid_core, pl.ds(rlo, P)])


@functools.partial(jax.jit, static_argnames=())
def _sc_scatter(xp, meta, zeros):
    mesh = plsc.VectorSubcoreMesh(core_axis_name="c", subcore_axis_name="s")
    fn = pl.kernel(
        _sc_kernel_body,
            out_type=jax.ShapeDtypeStruct((2, HALF, CP), jnp.float32),
        mesh=mesh,
        scratch_types=[
            pltpu.VMEM((4, P), jnp.float32),
            pltpu.VMEM((P,), jnp.int32),
            pltpu.VMEM((P,), jnp.int32),
            pltpu.VMEM((P, CP), jnp.float32),
            pltpu.VMEM((P, CP), jnp.float32),
            pltpu.SemaphoreType.DMA,
            pltpu.SemaphoreType.DMA,
            pltpu.VMEM_SHARED((GRID_ROWS, CP), jnp.float32),
        ],
    )
    return fn(xp, meta, zeros)


def _tc_merge_body(p_ref, o_ref):
    t = p_ref[0]
    o_ref[...] = t[:, :C].T


def _tc_merge(partials):
    BLK = 128
    nblk = (NX * NY) // BLK
    hblk = HALF // BLK
    return pl.pallas_call(
        _tc_merge_body,
        grid=(nblk,),
        in_specs=[pl.BlockSpec((1, BLK, CP),
                               lambda i: (i // hblk, i % hblk, 0))],
        out_specs=pl.BlockSpec((C, BLK), lambda i: (0, i)),
        out_shape=jax.ShapeDtypeStruct((C, NX * NY), jnp.float32),
    )(partials)


def kernel(x, geom, depth_kept):
    xp = jnp.pad(x.reshape(NCHUNKS, P, C), ((0, 0), (0, 0), (0, CP - C)))
    geom_t = geom.reshape(NPRIME, 3).T
    dk = depth_kept.reshape(1, NPRIME).astype(jnp.float32)
    meta = jnp.concatenate([geom_t, dk], axis=0)
    meta = meta.reshape(4, NCHUNKS, P).transpose(1, 0, 2)
    zeros = jnp.zeros((P, CP), jnp.float32)
    partials = _sc_scatter(xp, meta, zeros)
    out = _tc_merge(partials)
    return out.reshape(B, NZ * C, NX, NY)
